# baseline (speedup 1.0000x reference)
import jax
import jax.numpy as jnp
from jax.experimental import pallas as pl
from jax.experimental.pallas import tpu as pltpu

_BR = 4
_LOOK = 8
_D = _BR * (_LOOK + 1)
_BM = 64


def _height_body(idx_ref, table_ref, lat_ref, out_ref, wbuf, sems):
    i = pl.program_id(0)
    n_steps = pl.num_programs(0)

    def start(r, slot):
        pltpu.make_async_copy(
            table_ref.at[pl.ds(idx_ref[r], 1)],
            wbuf.at[pl.ds(slot, 1)],
            sems.at[slot],
        ).start()

    @pl.when(i == 0)
    def _():
        for r in range(_BR * _LOOK):
            start(r, r % _D)

    @pl.when(i + _LOOK < n_steps)
    def _():
        for j in range(_BR):
            r = (i + _LOOK) * _BR + j
            start(r, lax.rem(r, _D))

    lat = lat_ref[...]
    for j in range(_BR):
        r = i * _BR + j
        slot = lax.rem(r, _D)
        pltpu.make_async_copy(
            table_ref.at[pl.ds(idx_ref[r], 1)],
            wbuf.at[pl.ds(slot, 1)],
            sems.at[slot],
        ).wait()
        w = wbuf[pl.ds(slot, 1)][0]
        out_ref[:, j * 32:(j + 1) * 32] = jnp.dot(
            lat, w, preferred_element_type=jnp.float32
        )


def _baseline_body(lat_ref, wbt_ref, out_ref):
    out_ref[...] = jnp.dot(
        lat_ref[...], wbt_ref[...], preferred_element_type=jnp.float32
    )


from jax import lax


def kernel(latent, regions_oi, delta_height_weight, delta_baseline_weight):
    n_cells, n_latent = latent.shape
    n_regions, _, n_comp = delta_height_weight.shape
    n_oi = regions_oi.shape[0]
    n_flat = n_oi * n_comp

    grid_spec = pltpu.PrefetchScalarGridSpec(
        num_scalar_prefetch=1,
        grid=(n_oi // _BR,),
        in_specs=[
            pl.BlockSpec(memory_space=pl.ANY),
            pl.BlockSpec((n_cells, n_latent), lambda i, idx_ref: (0, 0)),
        ],
        out_specs=pl.BlockSpec(
            (n_cells, _BR * n_comp), lambda i, idx_ref: (0, i)
        ),
        scratch_shapes=[
            pltpu.VMEM((_D, n_latent, n_comp), jnp.float32),
            pltpu.SemaphoreType.DMA((_D,)),
        ],
    )
    h2 = pl.pallas_call(
        _height_body,
        grid_spec=grid_spec,
        out_shape=jax.ShapeDtypeStruct((n_cells, n_flat), jnp.float32),
    )(regions_oi, delta_height_weight, latent)
    delta_height = h2.reshape(n_cells, n_oi, n_comp)

    n_full = delta_baseline_weight.shape[0]
    wbt = delta_baseline_weight.T
    delta_baseline = pl.pallas_call(
        _baseline_body,
        grid=(n_cells // _BM,),
        in_specs=[
            pl.BlockSpec((_BM, n_latent), lambda m: (m, 0)),
            pl.BlockSpec((n_latent, n_full), lambda m: (0, 0)),
        ],
        out_specs=pl.BlockSpec((_BM, n_full), lambda m: (m, 0)),
        out_shape=jax.ShapeDtypeStruct((n_cells, n_full), jnp.float32),
    )(latent, wbt)

    return (delta_height, delta_baseline)
